# baseline (speedup 1.0000x reference)
import functools

import jax
import jax.numpy as jnp
from jax import lax
from jax.experimental import pallas as pl
from jax.experimental.pallas import tpu as pltpu
from jax.experimental.pallas import tpu_sc as plsc

NW = 32
CHUNK = 256
DPAD = 128


def _gather_kernel(B, V, D):
    b_per_w = B // NW
    n_chunks = b_per_w // CHUNK
    mesh = plsc.VectorSubcoreMesh(core_axis_name="c", subcore_axis_name="s")

    scratch = (
        [pltpu.VMEM((CHUNK,), jnp.int32) for _ in range(2)]
        + [pltpu.VMEM((CHUNK, DPAD), jnp.float32) for _ in range(2)]
        + [pltpu.VMEM((D, CHUNK), jnp.float32) for _ in range(2)]
        + [pltpu.SemaphoreType.DMA for _ in range(6)]
    )

    @functools.partial(
        pl.kernel,
        out_type=jax.ShapeDtypeStruct((D, B), jnp.float32),
        mesh=mesh,
        scratch_types=scratch,
        compiler_params=pltpu.CompilerParams(use_tc_tiling_on_sc=True, needs_layout_passes=False),
    )
    def k(idx_hbm, table_hbm, out_hbm, *refs):
        idx_bufs = refs[0:2]
        rows_bufs = refs[2:4]
        tr_bufs = refs[4:6]
        sem_i = refs[6:8]
        sem_g = refs[8:10]
        sem_o = refs[10:12]

        nc = 2
        wid = lax.axis_index("s") * nc + lax.axis_index("c")
        base = wid * b_per_w

        def issue_idx(g, b):
            pltpu.async_copy(
                idx_hbm.at[pl.ds(base + g * CHUNK, CHUNK)], idx_bufs[b], sem_i[b]
            )

        def wait_idx(b):
            pltpu.make_async_copy(
                idx_hbm.at[pl.ds(0, CHUNK)], idx_bufs[b], sem_i[b]
            ).wait()

        def issue_gather(b):
            pltpu.async_copy(table_hbm.at[idx_bufs[b]], rows_bufs[b], sem_g[b])

        def wait_gather(b):
            pltpu.make_async_copy(
                table_hbm.at[idx_bufs[b]], rows_bufs[b], sem_g[b]
            ).wait()

        def issue_out(g, b):
            pltpu.async_copy(
                tr_bufs[b],
                out_hbm.at[:, pl.ds(base + g * CHUNK, CHUNK)],
                sem_o[b],
            )

        def wait_out(b):
            pltpu.make_async_copy(
                tr_bufs[b], out_hbm.at[:, pl.ds(0, CHUNK)], sem_o[b]
            ).wait()

        def transpose_chunk(rows_buf, tr_buf):
            @plsc.parallel_loop(0, D, 1, unroll=16)
            def dbody(d):
                col = jnp.full((16,), d, dtype=jnp.int32)
                for c0 in range(0, CHUNK, 16):
                    rows16 = lax.iota(jnp.int32, 16) + c0
                    v = plsc.load_gather(rows_buf, [rows16, col])
                    tr_buf[d, pl.ds(c0, 16)] = v

        issue_idx(0, 0)
        wait_idx(0)
        issue_gather(0)
        issue_idx(1, 1)

        def body(g, carry):
            b = lax.rem(g, 2)
            @pl.when(g + 1 < n_chunks)
            def _():
                for bb in range(2):
                    @pl.when(b != bb)
                    def _():
                        wait_idx(bb)
                        issue_gather(bb)

            @pl.when(g + 2 < n_chunks)
            def _():
                for bb in range(2):
                    @pl.when(b == bb)
                    def _():
                        wait_gather(bb)
                        issue_idx(g + 2, bb)

            for bb in range(2):
                @pl.when(b == bb)
                def _():
                    @pl.when(g + 2 >= n_chunks)
                    def _():
                        wait_gather(bb)
                    @pl.when(g >= 2)
                    def _():
                        wait_out(bb)
                    transpose_chunk(rows_bufs[bb], tr_bufs[bb])
                    issue_out(g, bb)
            return carry

        lax.fori_loop(0, n_chunks, body, 0)

        for bb in range(2):
            wait_out(bb)

    return k


def kernel(patch_indices, patch_embeddings):
    B = patch_indices.shape[0]
    V, D = patch_embeddings.shape
    idx = patch_indices.astype(jnp.int32)
    table_pad = jnp.pad(patch_embeddings, ((0, 0), (0, DPAD - D)))
    out_t = _gather_kernel(B, V, D)(idx, table_pad)
    return out_t.T


# speedup vs baseline: 1.0465x; 1.0465x over previous
import functools

import jax
import jax.numpy as jnp
from jax import lax
from jax.experimental import pallas as pl
from jax.experimental.pallas import tpu as pltpu
from jax.experimental.pallas import tpu_sc as plsc

NW = 32
NBUF = 8
CHUNK = 128


def _gather_kernel(B, V, D):
    b_per_w = B // NW
    n_chunks = b_per_w // CHUNK
    n_outer = n_chunks // NBUF
    mesh = plsc.VectorSubcoreMesh(core_axis_name="c", subcore_axis_name="s")

    scratch = (
        [pltpu.VMEM((CHUNK,), jnp.int32) for _ in range(NBUF)]
        + [pltpu.VMEM((CHUNK, D), jnp.float32) for _ in range(NBUF)]
        + [pltpu.SemaphoreType.DMA for _ in range(3 * NBUF)]
    )

    @functools.partial(
        pl.kernel,
        out_type=jax.ShapeDtypeStruct((B, D), jnp.float32),
        mesh=mesh,
        scratch_types=scratch,
        compiler_params=pltpu.CompilerParams(use_tc_tiling_on_sc=False),
    )
    def k(idx_hbm, table_hbm, out_hbm, *scratch_refs):
        idx_bufs = scratch_refs[:NBUF]
        rows_bufs = scratch_refs[NBUF : 2 * NBUF]
        sem_i = scratch_refs[2 * NBUF : 3 * NBUF]
        sem_g = scratch_refs[3 * NBUF : 4 * NBUF]
        sem_o = scratch_refs[4 * NBUF : 5 * NBUF]

        nc = 2
        wid = lax.axis_index("s") * nc + lax.axis_index("c")
        base = wid * b_per_w

        def issue_idx(g, b):
            pltpu.async_copy(
                idx_hbm.at[pl.ds(base + g * CHUNK, CHUNK)], idx_bufs[b], sem_i[b]
            )

        def wait_idx(b):
            pltpu.make_async_copy(
                idx_hbm.at[pl.ds(0, CHUNK)], idx_bufs[b], sem_i[b]
            ).wait()

        def issue_gather(b):
            pltpu.async_copy(table_hbm.at[idx_bufs[b]], rows_bufs[b], sem_g[b])

        def wait_gather(b):
            pltpu.make_async_copy(
                table_hbm.at[idx_bufs[b]], rows_bufs[b], sem_g[b]
            ).wait()

        def issue_out(g, b):
            pltpu.async_copy(
                rows_bufs[b], out_hbm.at[pl.ds(base + g * CHUNK, CHUNK)], sem_o[b]
            )

        def wait_out(b):
            pltpu.make_async_copy(
                rows_bufs[b], out_hbm.at[pl.ds(0, CHUNK)], sem_o[b]
            ).wait()

        for b in range(NBUF):
            issue_idx(b, b)
        for b in range(NBUF):
            wait_idx(b)
            issue_gather(b)

        def body(t, carry):
            g0 = t * NBUF
            for b in range(NBUF):
                wait_gather(b)
                issue_out(g0 - NBUF + b, b)
            for b in range(NBUF):
                issue_idx(g0 + b, b)
            for b in range(NBUF):
                wait_out(b)
                wait_idx(b)
                issue_gather(b)
            return carry

        lax.fori_loop(1, n_outer, body, 0)

        g0 = n_outer * NBUF
        for b in range(NBUF):
            wait_gather(b)
            issue_out(g0 - NBUF + b, b)
        for b in range(NBUF):
            wait_out(b)

    return k


def kernel(patch_indices, patch_embeddings):
    B = patch_indices.shape[0]
    V, D = patch_embeddings.shape
    idx = patch_indices.astype(jnp.int32)
    return _gather_kernel(B, V, D)(idx, patch_embeddings)


# speedup vs baseline: 1.3578x; 1.2974x over previous
import functools

import jax
import jax.numpy as jnp
from jax import lax
from jax.experimental import pallas as pl
from jax.experimental.pallas import tpu as pltpu
from jax.experimental.pallas import tpu_sc as plsc

NW = 32
NBUF = 4
CHUNK = 128


def _gather_kernel(B, V, D):
    b_per_w = B // NW
    n_chunks = b_per_w // CHUNK
    n_outer = n_chunks // NBUF
    mesh = plsc.VectorSubcoreMesh(core_axis_name="c", subcore_axis_name="s")

    DPAD = 128
    scratch = (
        [pltpu.VMEM((CHUNK,), jnp.int32) for _ in range(NBUF)]
        + [pltpu.VMEM((CHUNK, DPAD), jnp.float32) for _ in range(NBUF)]
        + [pltpu.SemaphoreType.DMA for _ in range(3 * NBUF)]
    )

    @functools.partial(
        pl.kernel,
        out_type=jax.ShapeDtypeStruct((B, 128), jnp.float32),
        mesh=mesh,
        scratch_types=scratch,
        compiler_params=pltpu.CompilerParams(use_tc_tiling_on_sc=True, needs_layout_passes=False),
    )
    def k(idx_hbm, table_hbm, out_hbm, *scratch_refs):
        idx_bufs = scratch_refs[:NBUF]
        rows_bufs = scratch_refs[NBUF : 2 * NBUF]
        sem_i = scratch_refs[2 * NBUF : 3 * NBUF]
        sem_g = scratch_refs[3 * NBUF : 4 * NBUF]
        sem_o = scratch_refs[4 * NBUF : 5 * NBUF]

        nc = 2
        wid = lax.axis_index("s") * nc + lax.axis_index("c")
        base = wid * b_per_w

        def issue_idx(g, b):
            pltpu.async_copy(
                idx_hbm.at[pl.ds(base + g * CHUNK, CHUNK)], idx_bufs[b], sem_i[b]
            )

        def wait_idx(b):
            pltpu.make_async_copy(
                idx_hbm.at[pl.ds(0, CHUNK)], idx_bufs[b], sem_i[b]
            ).wait()

        def issue_gather(b):
            pltpu.async_copy(table_hbm.at[idx_bufs[b]], rows_bufs[b], sem_g[b])

        def wait_gather(b):
            pltpu.make_async_copy(
                table_hbm.at[idx_bufs[b]], rows_bufs[b], sem_g[b]
            ).wait()

        def issue_out(g, b):
            pltpu.async_copy(
                rows_bufs[b], out_hbm.at[pl.ds(base + g * CHUNK, CHUNK)], sem_o[b]
            )

        def wait_out(b):
            pltpu.make_async_copy(
                rows_bufs[b], out_hbm.at[pl.ds(0, CHUNK)], sem_o[b]
            ).wait()

        for b in range(NBUF):
            issue_idx(b, b)
        for b in range(NBUF):
            wait_idx(b)
            issue_gather(b)

        def body(t, carry):
            g0 = t * NBUF
            for b in range(NBUF):
                wait_gather(b)
                issue_out(g0 - NBUF + b, b)
            for b in range(NBUF):
                issue_idx(g0 + b, b)
            for b in range(NBUF):
                wait_out(b)
                wait_idx(b)
                issue_gather(b)
            return carry

        lax.fori_loop(1, n_outer, body, 0)

        g0 = n_outer * NBUF
        for b in range(NBUF):
            wait_gather(b)
            issue_out(g0 - NBUF + b, b)
        for b in range(NBUF):
            wait_out(b)

    return k


def kernel(patch_indices, patch_embeddings):
    B = patch_indices.shape[0]
    V, D = patch_embeddings.shape
    idx = patch_indices.astype(jnp.int32)
    table_pad = jnp.pad(patch_embeddings, ((0, 0), (0, 128 - D)))
    return _gather_kernel(B, V, D)(idx, table_pad)[:, :D]


# speedup vs baseline: 1.3642x; 1.0047x over previous
import functools

import jax
import jax.numpy as jnp
from jax import lax
from jax.experimental import pallas as pl
from jax.experimental.pallas import tpu as pltpu
from jax.experimental.pallas import tpu_sc as plsc

NW = 32
NBUF = 2
CHUNK = 256


def _gather_kernel(B, V, D):
    b_per_w = B // NW
    n_chunks = b_per_w // CHUNK
    n_outer = n_chunks // NBUF
    mesh = plsc.VectorSubcoreMesh(core_axis_name="c", subcore_axis_name="s")

    DPAD = 128
    scratch = (
        [pltpu.VMEM((CHUNK,), jnp.int32) for _ in range(NBUF)]
        + [pltpu.VMEM((CHUNK, DPAD), jnp.float32) for _ in range(NBUF)]
        + [pltpu.SemaphoreType.DMA for _ in range(3 * NBUF)]
    )

    @functools.partial(
        pl.kernel,
        out_type=jax.ShapeDtypeStruct((B, 128), jnp.float32),
        mesh=mesh,
        scratch_types=scratch,
        compiler_params=pltpu.CompilerParams(use_tc_tiling_on_sc=True, needs_layout_passes=False),
    )
    def k(idx_hbm, table_hbm, out_hbm, *scratch_refs):
        idx_bufs = scratch_refs[:NBUF]
        rows_bufs = scratch_refs[NBUF : 2 * NBUF]
        sem_i = scratch_refs[2 * NBUF : 3 * NBUF]
        sem_g = scratch_refs[3 * NBUF : 4 * NBUF]
        sem_o = scratch_refs[4 * NBUF : 5 * NBUF]

        nc = 2
        wid = lax.axis_index("s") * nc + lax.axis_index("c")
        base = wid * b_per_w

        def issue_idx(g, b):
            pltpu.async_copy(
                idx_hbm.at[pl.ds(base + g * CHUNK, CHUNK)], idx_bufs[b], sem_i[b]
            )

        def wait_idx(b):
            pltpu.make_async_copy(
                idx_hbm.at[pl.ds(0, CHUNK)], idx_bufs[b], sem_i[b]
            ).wait()

        def issue_gather(b):
            pltpu.async_copy(table_hbm.at[idx_bufs[b]], rows_bufs[b], sem_g[b])

        def wait_gather(b):
            pltpu.make_async_copy(
                table_hbm.at[idx_bufs[b]], rows_bufs[b], sem_g[b]
            ).wait()

        def issue_out(g, b):
            pltpu.async_copy(
                rows_bufs[b], out_hbm.at[pl.ds(base + g * CHUNK, CHUNK)], sem_o[b]
            )

        def wait_out(b):
            pltpu.make_async_copy(
                rows_bufs[b], out_hbm.at[pl.ds(0, CHUNK)], sem_o[b]
            ).wait()

        for b in range(NBUF):
            issue_idx(b, b)
        for b in range(NBUF):
            wait_idx(b)
            issue_gather(b)

        def body(t, carry):
            g0 = t * NBUF
            for b in range(NBUF):
                wait_gather(b)
                issue_out(g0 - NBUF + b, b)
            for b in range(NBUF):
                issue_idx(g0 + b, b)
            for b in range(NBUF):
                wait_out(b)
                wait_idx(b)
                issue_gather(b)
            return carry

        lax.fori_loop(1, n_outer, body, 0)

        g0 = n_outer * NBUF
        for b in range(NBUF):
            wait_gather(b)
            issue_out(g0 - NBUF + b, b)
        for b in range(NBUF):
            wait_out(b)

    return k


def kernel(patch_indices, patch_embeddings):
    B = patch_indices.shape[0]
    V, D = patch_embeddings.shape
    idx = patch_indices.astype(jnp.int32)
    table_pad = jnp.pad(patch_embeddings, ((0, 0), (0, 128 - D)))
    return _gather_kernel(B, V, D)(idx, table_pad)[:, :D]
